# Initial kernel scaffold; baseline (speedup 1.0000x reference)
import jax
import jax.numpy as jnp
from jax.experimental import pallas as pl


def kernel(x, table, training):
    raise NotImplementedError("write your pallas kernel here")


# speedup vs baseline: 1.0858x; 1.0858x over previous
import functools

import jax
import jax.numpy as jnp
from jax import lax
from jax.experimental import pallas as pl
from jax.experimental.pallas import tpu as pltpu
from jax.experimental.pallas import tpu_sc as plsc

BATCH = 16384
HIST = 50
DIM = 64

NC = 2
NS = 16
NW = NC * NS

B_PER_W = BATCH // NW
IDX_PER_W = B_PER_W * HIST
BLK_B = 4
ROWS_BLK = BLK_B * HIST
GCHUNK = 40
NBLK = B_PER_W // BLK_B
OUT_BLK = BLK_B * DIM * HIST


def _build_sc_call():
    mesh = plsc.VectorSubcoreMesh(core_axis_name="c", subcore_axis_name="s")

    @functools.partial(
        pl.kernel,
        mesh=mesh,
        compiler_params=pltpu.CompilerParams(
            needs_layout_passes=False, use_tc_tiling_on_sc=False
        ),
        out_type=jax.ShapeDtypeStruct((BATCH * DIM * HIST,), jnp.float32),
        scratch_types=[
            pltpu.VMEM((IDX_PER_W,), jnp.int32),
            pltpu.VMEM((ROWS_BLK, DIM), jnp.float32),
            pltpu.VMEM((OUT_BLK,), jnp.float32),
            pltpu.SemaphoreType.DMA,
        ],
    )
    def emb_kernel(x_hbm, table_hbm, out_hbm, idx_v, rows_v, outt_v, gsem):
        wid = lax.axis_index("s") * NC + lax.axis_index("c")
        pltpu.sync_copy(x_hbm.at[pl.ds(wid * IDX_PER_W, IDX_PER_W)], idx_v)
        lane_l = lax.iota(jnp.int32, 16) * HIST

        def blk(g, carry):
            cps = [
                pltpu.async_copy(
                    table_hbm.at[idx_v.at[pl.ds(g * ROWS_BLK + c * GCHUNK, GCHUNK)]],
                    rows_v.at[pl.ds(c * GCHUNK, GCHUNK)],
                    gsem,
                )
                for c in range(ROWS_BLK // GCHUNK)
            ]
            for cp in cps:
                cp.wait()

            for b in range(BLK_B):
                def ll_body(ll, c2, b=b):
                    r = b * HIST + ll
                    for c in range(DIM // 16):
                        v = rows_v[r, pl.ds(c * 16, 16)]
                        sidx = lane_l + (b * DIM * HIST + c * 16 * HIST + ll)
                        plsc.store_scatter(outt_v, [sidx], v)
                    return c2
                lax.fori_loop(0, HIST, ll_body, 0)

            pltpu.sync_copy(
                outt_v,
                out_hbm.at[pl.ds((wid * B_PER_W + g * BLK_B) * DIM * HIST, OUT_BLK)],
            )
            return carry

        lax.fori_loop(0, NBLK, blk, 0)

    return emb_kernel


_emb = _build_sc_call()


def kernel(x, table, training):
    del training
    x_flat = x.reshape(-1).astype(jnp.int32)
    out = _emb(x_flat, table)
    return out.reshape(BATCH, DIM, HIST)


# speedup vs baseline: 1.2125x; 1.1166x over previous
import functools

import jax
import jax.numpy as jnp
from jax import lax
from jax.experimental import pallas as pl
from jax.experimental.pallas import tpu as pltpu
from jax.experimental.pallas import tpu_sc as plsc

BATCH = 16384
HIST = 50
DIM = 64

NC = 2
NS = 16
NW = NC * NS

B_PER_W = BATCH // NW
IDX_PER_W = B_PER_W * HIST
BLK_B = 4
ROWS_BLK = BLK_B * HIST
GCHUNK = 40
NBLK = B_PER_W // BLK_B
OUT_BLK = BLK_B * DIM * HIST


def _build_sc_call():
    mesh = plsc.VectorSubcoreMesh(core_axis_name="c", subcore_axis_name="s")

    @functools.partial(
        pl.kernel,
        mesh=mesh,
        compiler_params=pltpu.CompilerParams(
            needs_layout_passes=False, use_tc_tiling_on_sc=False
        ),
        out_type=jax.ShapeDtypeStruct((BATCH, DIM, HIST), jnp.float32),
        scratch_types=[
            pltpu.VMEM((IDX_PER_W,), jnp.int32),
            pltpu.VMEM((ROWS_BLK, DIM), jnp.float32),
            pltpu.VMEM((BLK_B, DIM, HIST), jnp.float32),
            pltpu.SemaphoreType.DMA,
        ],
    )
    def emb_kernel(x_hbm, table_hbm, out_hbm, idx_v, rows_v, outt_v, gsem):
        wid = lax.axis_index("s") * NC + lax.axis_index("c")
        pltpu.sync_copy(x_hbm.at[pl.ds(wid * IDX_PER_W, IDX_PER_W)], idx_v)
        lane16 = lax.iota(jnp.int32, 16)

        def blk(g, carry):
            cps = [
                pltpu.async_copy(
                    table_hbm.at[idx_v.at[pl.ds(g * ROWS_BLK + c * GCHUNK, GCHUNK)]],
                    rows_v.at[pl.ds(c * GCHUNK, GCHUNK)],
                    gsem,
                )
                for c in range(ROWS_BLK // GCHUNK)
            ]
            for cp in cps:
                cp.wait()

            for b in range(BLK_B):
                idx_b = lax.broadcast(jnp.int32(b), (16,))

                def ll_body(ll, c2, b=b, idx_b=idx_b):
                    r = b * HIST + ll
                    idx_l = lax.broadcast(ll, (16,))
                    for c in range(DIM // 16):
                        v = rows_v[r, pl.ds(c * 16, 16)]
                        plsc.store_scatter(
                            outt_v, [idx_b, lane16 + c * 16, idx_l], v
                        )
                    return c2
                lax.fori_loop(0, HIST, ll_body, 0)

            pltpu.sync_copy(
                outt_v,
                out_hbm.at[pl.ds(wid * B_PER_W + g * BLK_B, BLK_B)],
            )
            return carry

        lax.fori_loop(0, NBLK, blk, 0)

    return emb_kernel


_emb = _build_sc_call()


def kernel(x, table, training):
    del training
    x_flat = x.reshape(-1).astype(jnp.int32)
    return _emb(x_flat, table)


# speedup vs baseline: 1.2813x; 1.0568x over previous
import functools

import jax
import jax.numpy as jnp
from jax import lax
from jax.experimental import pallas as pl
from jax.experimental.pallas import tpu as pltpu
from jax.experimental.pallas import tpu_sc as plsc

BATCH = 16384
HIST = 50
DIM = 64

NC = 2
NS = 16
NW = NC * NS

B_PER_W = BATCH // NW
IDX_PER_W = B_PER_W * HIST
BT_PER_W = B_PER_W // 128
NSTEP = BT_PER_W * HIST


def _build_sc_call():
    mesh = plsc.VectorSubcoreMesh(core_axis_name="c", subcore_axis_name="s")

    @functools.partial(
        pl.kernel,
        mesh=mesh,
        compiler_params=pltpu.CompilerParams(
            needs_layout_passes=False, use_tc_tiling_on_sc=False
        ),
        out_type=jax.ShapeDtypeStruct(
            (HIST, DIM // 8, BATCH // 128, 8, 128), jnp.float32
        ),
        scratch_types=[
            pltpu.VMEM((IDX_PER_W,), jnp.int32),
            pltpu.VMEM((128,), jnp.int32),
            pltpu.VMEM((2, 128, DIM), jnp.float32),
            pltpu.VMEM((2, DIM, 128), jnp.float32),
            pltpu.SemaphoreType.DMA,
            pltpu.SemaphoreType.DMA,
            pltpu.SemaphoreType.DMA,
        ],
    )
    def emb_kernel(
        x_hbm, table_hbm, out_hbm, idx_v, glist_v, rows_v, blk_v,
        gsem, osem0, osem1,
    ):
        wid = lax.axis_index("s") * NC + lax.axis_index("c")
        pltpu.sync_copy(x_hbm.at[pl.ds(wid * IDX_PER_W, IDX_PER_W)], idx_v)

        lane16 = lax.iota(jnp.int32, 16)
        iota50 = lane16 * HIST
        rowidx = [lane16 + 16 * k for k in range(8)]
        osems = (osem0, osem1)

        def extract(step):
            btl = step // HIST
            l = step % HIST
            base = btl * (128 * HIST) + l
            for k in range(8):
                g = plsc.load_gather(idx_v, [iota50 + (base + k * 16 * HIST)])
                glist_v[pl.ds(16 * k, 16)] = g

        def start_gather(buf):
            return pltpu.async_copy(
                table_hbm.at[glist_v], rows_v.at[buf], gsem
            )

        def body(t, cur):
            btl = t // HIST
            l = t % HIST
            nxt = 1 - cur

            @pl.when(t >= 2)
            def _():
                pltpu.make_async_copy(
                    out_hbm.at[0, 0, 0], blk_v.at[cur, pl.ds(0, 8)], osems[cur]
                ).wait()
                pltpu.make_async_copy(
                    out_hbm.at[0, 0, 0], blk_v.at[cur, pl.ds(8, 8)], osems[cur]
                ).wait()
                pltpu.make_async_copy(
                    out_hbm.at[0, 0, 0], blk_v.at[cur, pl.ds(16, 8)], osems[cur]
                ).wait()
                pltpu.make_async_copy(
                    out_hbm.at[0, 0, 0], blk_v.at[cur, pl.ds(24, 8)], osems[cur]
                ).wait()
                pltpu.make_async_copy(
                    out_hbm.at[0, 0, 0], blk_v.at[cur, pl.ds(32, 8)], osems[cur]
                ).wait()
                pltpu.make_async_copy(
                    out_hbm.at[0, 0, 0], blk_v.at[cur, pl.ds(40, 8)], osems[cur]
                ).wait()
                pltpu.make_async_copy(
                    out_hbm.at[0, 0, 0], blk_v.at[cur, pl.ds(48, 8)], osems[cur]
                ).wait()
                pltpu.make_async_copy(
                    out_hbm.at[0, 0, 0], blk_v.at[cur, pl.ds(56, 8)], osems[cur]
                ).wait()

            extract(jnp.minimum(t + 1, NSTEP - 1))
            cp = start_gather(nxt)

            rflat = rows_v.at[cur]

            def d_body(d, c2):
                d_bc = lane16 * 0 + d
                for k in range(8):
                    v = plsc.load_gather(rflat, [rowidx[k], d_bc])
                    blk_v[cur, d, pl.ds(16 * k, 16)] = v
                return c2

            lax.fori_loop(0, DIM, d_body, 0, unroll=2)

            bt = wid * BT_PER_W + btl
            for dt in range(8):
                pltpu.async_copy(
                    blk_v.at[cur, pl.ds(dt * 8, 8)],
                    out_hbm.at[l, dt, bt],
                    osems[cur],
                )

            cp.wait()
            return cur

        extract(jnp.int32(0))
        start_gather(0).wait()

        def pair(g, carry):
            body(2 * g, 0)
            body(2 * g + 1, 1)
            return carry

        lax.fori_loop(0, NSTEP // 2, pair, 0)

        for p in range(2):
            for dt in range(8):
                pltpu.make_async_copy(
                    out_hbm.at[0, 0, 0],
                    blk_v.at[p, pl.ds(dt * 8, 8)],
                    osems[p],
                ).wait()

    return emb_kernel


_emb = _build_sc_call()


def kernel(x, table, training):
    del training
    x_flat = x.reshape(-1).astype(jnp.int32)
    out5 = _emb(x_flat, table)
    return out5.transpose(2, 4, 1, 3, 0).reshape(BATCH, DIM, HIST)


# speedup vs baseline: 2.3304x; 1.8187x over previous
import functools

import jax
import jax.numpy as jnp
from jax import lax
from jax.experimental import pallas as pl
from jax.experimental.pallas import tpu as pltpu
from jax.experimental.pallas import tpu_sc as plsc

BATCH = 16384
HIST = 50
DIM = 64

NC = 2
NS = 16
NW = NC * NS

B_PER_W = BATCH // NW
IDX_PER_W = B_PER_W * HIST
BT_PER_W = B_PER_W // 128
NSTEP = BT_PER_W * HIST


def _build_sc_call():
    mesh = plsc.VectorSubcoreMesh(core_axis_name="c", subcore_axis_name="s")

    @functools.partial(
        pl.kernel,
        mesh=mesh,
        compiler_params=pltpu.CompilerParams(
            needs_layout_passes=False, use_tc_tiling_on_sc=False
        ),
        out_type=jax.ShapeDtypeStruct(
            (HIST, DIM // 8, BATCH // 128, 8, 128), jnp.float32
        ),
        scratch_types=[
            pltpu.VMEM((IDX_PER_W,), jnp.int32),
            pltpu.VMEM((128,), jnp.int32),
            pltpu.VMEM((2, 128, DIM), jnp.float32),
            pltpu.VMEM((2, DIM, 129), jnp.float32),
            pltpu.SemaphoreType.DMA,
            pltpu.SemaphoreType.DMA,
            pltpu.SemaphoreType.DMA,
        ],
    )
    def emb_kernel(
        x_hbm, table_hbm, out_hbm, idx_v, glist_v, rows_v, blk_v,
        gsem, osem0, osem1,
    ):
        wid = lax.axis_index("s") * NC + lax.axis_index("c")
        pltpu.sync_copy(x_hbm.at[pl.ds(wid * IDX_PER_W, IDX_PER_W)], idx_v)

        lane16 = lax.iota(jnp.int32, 16)
        iota50 = lane16 * HIST
        rowidx = [lane16 + 16 * k for k in range(8)]
        osems = (osem0, osem1)

        def extract(step):
            btl = step // HIST
            l = step % HIST
            base = btl * (128 * HIST) + l
            for k in range(8):
                g = plsc.load_gather(idx_v, [iota50 + (base + k * 16 * HIST)])
                glist_v[pl.ds(16 * k, 16)] = g

        def start_gather(buf):
            return pltpu.async_copy(
                table_hbm.at[glist_v], rows_v.at[buf], gsem
            )

        def body(t, cur):
            btl = t // HIST
            l = t % HIST
            nxt = 1 - cur

            @pl.when(t >= 2)
            def _():
                pltpu.make_async_copy(
                    out_hbm.at[0, 0, 0], blk_v.at[cur, pl.ds(0, 8), pl.ds(0, 128)], osems[cur]
                ).wait()
                pltpu.make_async_copy(
                    out_hbm.at[0, 0, 0], blk_v.at[cur, pl.ds(8, 8), pl.ds(0, 128)], osems[cur]
                ).wait()
                pltpu.make_async_copy(
                    out_hbm.at[0, 0, 0], blk_v.at[cur, pl.ds(16, 8), pl.ds(0, 128)], osems[cur]
                ).wait()
                pltpu.make_async_copy(
                    out_hbm.at[0, 0, 0], blk_v.at[cur, pl.ds(24, 8), pl.ds(0, 128)], osems[cur]
                ).wait()
                pltpu.make_async_copy(
                    out_hbm.at[0, 0, 0], blk_v.at[cur, pl.ds(32, 8), pl.ds(0, 128)], osems[cur]
                ).wait()
                pltpu.make_async_copy(
                    out_hbm.at[0, 0, 0], blk_v.at[cur, pl.ds(40, 8), pl.ds(0, 128)], osems[cur]
                ).wait()
                pltpu.make_async_copy(
                    out_hbm.at[0, 0, 0], blk_v.at[cur, pl.ds(48, 8), pl.ds(0, 128)], osems[cur]
                ).wait()
                pltpu.make_async_copy(
                    out_hbm.at[0, 0, 0], blk_v.at[cur, pl.ds(56, 8), pl.ds(0, 128)], osems[cur]
                ).wait()

            extract(jnp.minimum(t + 1, NSTEP - 1))
            cp = start_gather(nxt)

            rcur = rows_v.at[cur]
            bcur = blk_v.at[cur]

            def j_body(j, c2):
                j_bc = lane16 * 0 + j
                for c in range(DIM // 16):
                    v = rcur[j, pl.ds(16 * c, 16)]
                    plsc.store_scatter(bcur, [rowidx[c], j_bc], v)
                return c2

            lax.fori_loop(0, 128, j_body, 0, unroll=2)

            bt = wid * BT_PER_W + btl
            for dt in range(8):
                pltpu.async_copy(
                    blk_v.at[cur, pl.ds(dt * 8, 8), pl.ds(0, 128)],
                    out_hbm.at[l, dt, bt],
                    osems[cur],
                )

            cp.wait()
            return cur

        extract(jnp.int32(0))
        start_gather(0).wait()

        def pair(g, carry):
            body(2 * g, 0)
            body(2 * g + 1, 1)
            return carry

        lax.fori_loop(0, NSTEP // 2, pair, 0)

        for p in range(2):
            for dt in range(8):
                pltpu.make_async_copy(
                    out_hbm.at[0, 0, 0],
                    blk_v.at[p, pl.ds(dt * 8, 8), pl.ds(0, 128)],
                    osems[p],
                ).wait()

    return emb_kernel


_emb = _build_sc_call()


def kernel(x, table, training):
    del training
    x_flat = x.reshape(-1).astype(jnp.int32)
    out5 = _emb(x_flat, table)
    return out5.transpose(2, 4, 1, 3, 0).reshape(BATCH, DIM, HIST)


# speedup vs baseline: 2.3485x; 1.0078x over previous
import functools

import jax
import jax.numpy as jnp
from jax import lax
from jax.experimental import pallas as pl
from jax.experimental.pallas import tpu as pltpu
from jax.experimental.pallas import tpu_sc as plsc

BATCH = 16384
HIST = 50
DIM = 64

NC = 2
NS = 16
NW = NC * NS

B_PER_W = BATCH // NW
IDX_PER_W = B_PER_W * HIST
BT_PER_W = B_PER_W // 128
NSTEP = BT_PER_W * HIST


def _build_sc_call():
    mesh = plsc.VectorSubcoreMesh(core_axis_name="c", subcore_axis_name="s")

    @functools.partial(
        pl.kernel,
        mesh=mesh,
        compiler_params=pltpu.CompilerParams(
            needs_layout_passes=False, use_tc_tiling_on_sc=False
        ),
        out_type=jax.ShapeDtypeStruct(
            (HIST, DIM // 8, BATCH // 128, 8, 128), jnp.float32
        ),
        scratch_types=[
            pltpu.VMEM((IDX_PER_W,), jnp.int32),
            pltpu.VMEM((128,), jnp.int32),
            pltpu.VMEM((2, 128, DIM), jnp.float32),
            pltpu.VMEM((2, DIM, 129), jnp.float32),
            pltpu.SemaphoreType.DMA,
            pltpu.SemaphoreType.DMA,
            pltpu.SemaphoreType.DMA,
        ],
    )
    def emb_kernel(
        x_hbm, table_hbm, out_hbm, idx_v, glist_v, rows_v, blk_v,
        gsem, osem0, osem1,
    ):
        wid = lax.axis_index("s") * NC + lax.axis_index("c")
        pltpu.sync_copy(x_hbm.at[pl.ds(wid * IDX_PER_W, IDX_PER_W)], idx_v)

        lane16 = lax.iota(jnp.int32, 16)
        iota50 = lane16 * HIST
        rowidx = [lane16 + 16 * k for k in range(8)]
        osems = (osem0, osem1)

        def extract(step):
            btl = step // HIST
            l = step % HIST
            base = btl * (128 * HIST) + l
            for k in range(8):
                g = plsc.load_gather(idx_v, [iota50 + (base + k * 16 * HIST)])
                glist_v[pl.ds(16 * k, 16)] = g

        def start_gather(buf):
            return pltpu.async_copy(
                table_hbm.at[glist_v], rows_v.at[buf], gsem
            )

        def body(t, cur):
            btl = t // HIST
            l = t % HIST
            nxt = 1 - cur

            @pl.when(t >= 2)
            def _():
                pltpu.make_async_copy(
                    out_hbm.at[0, 0, 0], blk_v.at[cur, pl.ds(0, 8), pl.ds(0, 128)], osems[cur]
                ).wait()
                pltpu.make_async_copy(
                    out_hbm.at[0, 0, 0], blk_v.at[cur, pl.ds(8, 8), pl.ds(0, 128)], osems[cur]
                ).wait()
                pltpu.make_async_copy(
                    out_hbm.at[0, 0, 0], blk_v.at[cur, pl.ds(16, 8), pl.ds(0, 128)], osems[cur]
                ).wait()
                pltpu.make_async_copy(
                    out_hbm.at[0, 0, 0], blk_v.at[cur, pl.ds(24, 8), pl.ds(0, 128)], osems[cur]
                ).wait()
                pltpu.make_async_copy(
                    out_hbm.at[0, 0, 0], blk_v.at[cur, pl.ds(32, 8), pl.ds(0, 128)], osems[cur]
                ).wait()
                pltpu.make_async_copy(
                    out_hbm.at[0, 0, 0], blk_v.at[cur, pl.ds(40, 8), pl.ds(0, 128)], osems[cur]
                ).wait()
                pltpu.make_async_copy(
                    out_hbm.at[0, 0, 0], blk_v.at[cur, pl.ds(48, 8), pl.ds(0, 128)], osems[cur]
                ).wait()
                pltpu.make_async_copy(
                    out_hbm.at[0, 0, 0], blk_v.at[cur, pl.ds(56, 8), pl.ds(0, 128)], osems[cur]
                ).wait()

            extract(jnp.minimum(t + 1, NSTEP - 1))
            cp = start_gather(nxt)

            rcur = rows_v.at[cur]
            bcur = blk_v.at[cur]

            def j_body(j, c2):
                j_bc = lane16 * 0 + j
                for c in range(DIM // 16):
                    v = rcur[j, pl.ds(16 * c, 16)]
                    plsc.store_scatter(bcur, [rowidx[c], j_bc], v)
                return c2

            lax.fori_loop(0, 128, j_body, 0, unroll=4)

            bt = wid * BT_PER_W + btl
            for dt in range(8):
                pltpu.async_copy(
                    blk_v.at[cur, pl.ds(dt * 8, 8), pl.ds(0, 128)],
                    out_hbm.at[l, dt, bt],
                    osems[cur],
                )

            cp.wait()
            return cur

        extract(jnp.int32(0))
        start_gather(0).wait()

        def pair(g, carry):
            body(2 * g, 0)
            body(2 * g + 1, 1)
            return carry

        lax.fori_loop(0, NSTEP // 2, pair, 0)

        for p in range(2):
            for dt in range(8):
                pltpu.make_async_copy(
                    out_hbm.at[0, 0, 0],
                    blk_v.at[p, pl.ds(dt * 8, 8), pl.ds(0, 128)],
                    osems[p],
                ).wait()

    return emb_kernel


_emb = _build_sc_call()


def kernel(x, table, training):
    del training
    x_flat = x.reshape(-1).astype(jnp.int32)
    out5 = _emb(x_flat, table)
    return out5.transpose(2, 4, 1, 3, 0).reshape(BATCH, DIM, HIST)
